# baseline (speedup 1.0000x reference)
import jax
import jax.numpy as jnp
from jax.experimental import pallas as pl
from jax.experimental.pallas import tpu as pltpu

_B, _OBS, _H, _A, _E = 131072, 194, 64, 5, 3
_BB = 2048


def _fused_body(x_ref, act_ref, w1_ref, b1_ref, w2_ref, b2_ref, w3_ref, b3_ref,
                logp_ref, ent_ref, val_ref):
    x = x_ref[...]
    h = jnp.tanh(jnp.dot(x, w1_ref[...], preferred_element_type=jnp.float32)
                 + b1_ref[...])
    g = jnp.tanh(jnp.dot(h, w2_ref[...], preferred_element_type=jnp.float32)
                 + b2_ref[...])
    o = jnp.dot(g, w3_ref[...], preferred_element_type=jnp.float32) + b3_ref[...]

    x0, x1, x2 = x[:, 0:1], x[:, 1:2], x[:, 2:3]
    is0 = (x0 >= x1) & (x0 >= x2)
    is1 = jnp.logical_not(is0) & (x1 >= x2)
    logits = jnp.where(is0, o[:, 0:_A],
                       jnp.where(is1, o[:, _A:2 * _A], o[:, 2 * _A:3 * _A]))

    m = jnp.max(logits, axis=1, keepdims=True)
    z = logits - m
    ez = jnp.exp(z)
    se = jnp.sum(ez, axis=1, keepdims=True)
    lse = jnp.log(se)
    ent_ref[...] = jnp.sum((ez / se) * (lse - z), axis=1, keepdims=True)
    ai = jax.lax.broadcasted_iota(jnp.int32, (x.shape[0], _A), 1)
    sel = ai == act_ref[...]
    logp_ref[...] = jnp.sum(jnp.where(sel, z - lse, 0.0), axis=1, keepdims=True)
    val_ref[...] = o[:, 3 * _A:3 * _A + 1]


def kernel(x, action, W1, b1, W2, b2, Wh, bh, Wc1, bc1, Wc2, bc2, Wc3, bc3):
    B, OBS = x.shape
    H, A, E = _H, _A, _E

    w1c = jnp.concatenate([W1, Wc1], axis=1)
    b1c = jnp.concatenate([b1, bc1])[None, :]
    z64 = jnp.zeros((H, H), jnp.float32)
    w2c = jnp.block([[W2, z64], [z64, Wc2]])
    b2c = jnp.concatenate([b2, bc2])[None, :]
    wh_flat = jnp.transpose(Wh, (1, 0, 2)).reshape(H, E * A)
    w3c = jnp.concatenate([
        jnp.concatenate([wh_flat, jnp.zeros((H, 1), jnp.float32)], axis=1),
        jnp.concatenate([jnp.zeros((H, E * A), jnp.float32), Wc3], axis=1),
    ], axis=0)
    b3c = jnp.concatenate([bh.reshape(E * A), bc3])[None, :]

    act2d = action.astype(jnp.int32).reshape(B, 1)

    nb = B // _BB
    const = lambda i: (0, 0)
    out_shape = [jax.ShapeDtypeStruct((B, 1), jnp.float32)] * 3
    logp, ent, val = pl.pallas_call(
        _fused_body,
        grid=(nb,),
        in_specs=[
            pl.BlockSpec((_BB, OBS), lambda i: (i, 0)),
            pl.BlockSpec((_BB, 1), lambda i: (i, 0)),
            pl.BlockSpec((OBS, 2 * H), const),
            pl.BlockSpec((1, 2 * H), const),
            pl.BlockSpec((2 * H, 2 * H), const),
            pl.BlockSpec((1, 2 * H), const),
            pl.BlockSpec((2 * H, E * A + 1), const),
            pl.BlockSpec((1, E * A + 1), const),
        ],
        out_specs=[pl.BlockSpec((_BB, 1), lambda i: (i, 0))] * 3,
        out_shape=out_shape,
        compiler_params=pltpu.CompilerParams(
            dimension_semantics=("parallel",),
        ),
        name="agent_fused",
    )(x, act2d, w1c, b1c, w2c, b2c, w3c, b3c)

    return action, logp.reshape(B), ent.reshape(B), val


# speedup vs baseline: 5.4187x; 5.4187x over previous
import jax
import jax.numpy as jnp
from jax.experimental import pallas as pl
from jax.experimental.pallas import tpu as pltpu

_H, _A, _E = 64, 5, 3
_BB = 2048


def _fused_body(xt_ref, act_ref, w1_ref, b1_ref, w2_ref, b2_ref, w3_ref,
                b3_ref, logp_ref, ent_ref, val_ref):
    bb = xt_ref.shape[1]
    xt = xt_ref[...]
    h = jnp.tanh(jnp.dot(w1_ref[...], xt,
                         preferred_element_type=jnp.float32) + b1_ref[...])
    g = jnp.tanh(jnp.dot(w2_ref[...], h,
                         preferred_element_type=jnp.float32) + b2_ref[...])
    o = jnp.dot(w3_ref[...], g,
                preferred_element_type=jnp.float32) + b3_ref[...]

    x0, x1, x2 = xt[0:1, :], xt[1:2, :], xt[2:3, :]
    is0 = (x0 >= x1) & (x0 >= x2)
    is1 = jnp.logical_not(is0) & (x1 >= x2)
    ev5 = jnp.where(is0, 0, jnp.where(is1, _A, 2 * _A))

    ki = jax.lax.broadcasted_iota(jnp.int32, (_E * _A + 1, bb), 0)
    msel = (ki >= ev5) & (ki < ev5 + _A)
    mact = ki == ev5 + act_ref[...].reshape(1, bb)

    eo = jnp.exp(o)
    z1 = jnp.where(msel, eo, 0.0)
    se = jnp.sum(z1, axis=0, keepdims=True)
    sl = jnp.sum(z1 * o, axis=0, keepdims=True)
    la = jnp.sum(jnp.where(mact, o, 0.0), axis=0, keepdims=True)
    lse = jnp.log(se)

    logp_ref[...] = (la - lse).reshape(bb)
    ent_ref[...] = (lse - sl / se).reshape(bb)
    val_ref[...] = o[_E * _A:_E * _A + 1, :].reshape(bb)


def kernel(x, action, W1, b1, W2, b2, Wh, bh, Wc1, bc1, Wc2, bc2, Wc3, bc3):
    B, OBS = x.shape
    H, A, E = _H, _A, _E

    xt = x.T
    act = action.astype(jnp.int32)

    w1t = jnp.concatenate([W1.T, Wc1.T], axis=0)
    b1c = jnp.concatenate([b1, bc1])[:, None]
    z64 = jnp.zeros((H, H), jnp.float32)
    w2t = jnp.block([[W2.T, z64], [z64, Wc2.T]])
    b2c = jnp.concatenate([b2, bc2])[:, None]
    wh_t = jnp.transpose(Wh, (1, 0, 2)).reshape(H, E * A).T
    w3t = jnp.concatenate([
        jnp.concatenate([wh_t, jnp.zeros((E * A, H), jnp.float32)], axis=1),
        jnp.concatenate([jnp.zeros((1, H), jnp.float32), Wc3.T], axis=1),
    ], axis=0)
    b3c = jnp.concatenate([bh.reshape(E * A), bc3])[:, None]

    nb = B // _BB
    const = lambda i: (0, 0)
    out_shape = [jax.ShapeDtypeStruct((B,), jnp.float32)] * 3
    logp, ent, val = pl.pallas_call(
        _fused_body,
        grid=(nb,),
        in_specs=[
            pl.BlockSpec((OBS, _BB), lambda i: (0, i)),
            pl.BlockSpec((_BB,), lambda i: (i,)),
            pl.BlockSpec((2 * H, OBS), const),
            pl.BlockSpec((2 * H, 1), const),
            pl.BlockSpec((2 * H, 2 * H), const),
            pl.BlockSpec((2 * H, 1), const),
            pl.BlockSpec((E * A + 1, 2 * H), const),
            pl.BlockSpec((E * A + 1, 1), const),
        ],
        out_specs=[pl.BlockSpec((_BB,), lambda i: (i,))] * 3,
        out_shape=out_shape,
        compiler_params=pltpu.CompilerParams(
            dimension_semantics=("parallel",),
        ),
        name="agent_fused_t",
    )(xt, act, w1t, b1c, w2t, b2c, w3t, b3c)

    return action, logp, ent, val.reshape(B, 1)


# speedup vs baseline: 6.9718x; 1.2866x over previous
import jax
import jax.numpy as jnp
from jax.experimental import pallas as pl
from jax.experimental.pallas import tpu as pltpu

_H, _A, _E = 64, 5, 3
_BB = 4096


def _fused_body(xt_ref, act_ref, w1_ref, b1_ref, w2_ref, b2_ref, w3_ref,
                b3_ref, logp_ref, ent_ref, val_ref):
    bb = xt_ref.shape[1]
    xt = xt_ref[...]
    h = jnp.tanh(jnp.dot(w1_ref[...], xt,
                         preferred_element_type=jnp.float32) + b1_ref[...])
    g = jnp.tanh(jnp.dot(w2_ref[...], h,
                         preferred_element_type=jnp.float32) + b2_ref[...])
    o = jnp.dot(w3_ref[...], g,
                preferred_element_type=jnp.float32) + b3_ref[...]

    x0, x1, x2 = xt[0:1, :], xt[1:2, :], xt[2:3, :]
    is0 = (x0 >= x1) & (x0 >= x2)
    is1 = jnp.logical_not(is0) & (x1 >= x2)
    ev5 = jnp.where(is0, 0, jnp.where(is1, _A, 2 * _A))

    ki = jax.lax.broadcasted_iota(jnp.int32, (_E * _A + 1, bb), 0)
    msel = (ki >= ev5) & (ki < ev5 + _A)
    mact = ki == ev5 + act_ref[...].reshape(1, bb)

    eo = jnp.exp(o)
    z1 = jnp.where(msel, eo, 0.0)
    se = jnp.sum(z1, axis=0, keepdims=True)
    sl = jnp.sum(z1 * o, axis=0, keepdims=True)
    la = jnp.sum(jnp.where(mact, o, 0.0), axis=0, keepdims=True)
    lse = jnp.log(se)

    logp_ref[...] = (la - lse).reshape(bb)
    ent_ref[...] = (lse - sl / se).reshape(bb)
    val_ref[...] = o[_E * _A:_E * _A + 1, :].reshape(bb)


def kernel(x, action, W1, b1, W2, b2, Wh, bh, Wc1, bc1, Wc2, bc2, Wc3, bc3):
    B, OBS = x.shape
    H, A, E = _H, _A, _E

    xt = x.T
    act = action.astype(jnp.int32)

    w1t = jnp.concatenate([W1.T, Wc1.T], axis=0)
    b1c = jnp.concatenate([b1, bc1])[:, None]
    z64 = jnp.zeros((H, H), jnp.float32)
    w2t = jnp.block([[W2.T, z64], [z64, Wc2.T]])
    b2c = jnp.concatenate([b2, bc2])[:, None]
    wh_t = jnp.transpose(Wh, (1, 0, 2)).reshape(H, E * A).T
    w3t = jnp.concatenate([
        jnp.concatenate([wh_t, jnp.zeros((E * A, H), jnp.float32)], axis=1),
        jnp.concatenate([jnp.zeros((1, H), jnp.float32), Wc3.T], axis=1),
    ], axis=0)
    b3c = jnp.concatenate([bh.reshape(E * A), bc3])[:, None]

    nb = B // _BB
    const = lambda i: (0, 0)
    out_shape = [jax.ShapeDtypeStruct((B,), jnp.float32)] * 3
    logp, ent, val = pl.pallas_call(
        _fused_body,
        grid=(nb,),
        in_specs=[
            pl.BlockSpec((OBS, _BB), lambda i: (0, i)),
            pl.BlockSpec((_BB,), lambda i: (i,)),
            pl.BlockSpec((2 * H, OBS), const),
            pl.BlockSpec((2 * H, 1), const),
            pl.BlockSpec((2 * H, 2 * H), const),
            pl.BlockSpec((2 * H, 1), const),
            pl.BlockSpec((E * A + 1, 2 * H), const),
            pl.BlockSpec((E * A + 1, 1), const),
        ],
        out_specs=[pl.BlockSpec((_BB,), lambda i: (i,))] * 3,
        out_shape=out_shape,
        compiler_params=pltpu.CompilerParams(
            dimension_semantics=("parallel",),
        ),
        name="agent_fused_t",
    )(xt, act, w1t, b1c, w2t, b2c, w3t, b3c)

    return action, logp, ent, val.reshape(B, 1)


# speedup vs baseline: 8.2492x; 1.1832x over previous
import jax
import jax.numpy as jnp
from jax.experimental import pallas as pl
from jax.experimental.pallas import tpu as pltpu

_H, _A, _E = 64, 5, 3
_BB = 8192


def _fused_body(xt_ref, act_ref, w1_ref, b1_ref, w2_ref, b2_ref, w3_ref,
                b3_ref, logp_ref, ent_ref, val_ref):
    bb = xt_ref.shape[1]
    xt = xt_ref[...]
    h = jnp.tanh(jnp.dot(w1_ref[...], xt,
                         preferred_element_type=jnp.float32) + b1_ref[...])
    g = jnp.tanh(jnp.dot(w2_ref[...], h,
                         preferred_element_type=jnp.float32) + b2_ref[...])
    o = jnp.dot(w3_ref[...], g,
                preferred_element_type=jnp.float32) + b3_ref[...]

    x0, x1, x2 = xt[0:1, :], xt[1:2, :], xt[2:3, :]
    is0 = (x0 >= x1) & (x0 >= x2)
    is1 = jnp.logical_not(is0) & (x1 >= x2)
    ev5 = jnp.where(is0, 0, jnp.where(is1, _A, 2 * _A))

    ki = jax.lax.broadcasted_iota(jnp.int32, (_E * _A + 1, bb), 0)
    msel = (ki >= ev5) & (ki < ev5 + _A)
    mact = ki == ev5 + act_ref[...].reshape(1, bb)

    eo = jnp.exp(o)
    z1 = jnp.where(msel, eo, 0.0)
    se = jnp.sum(z1, axis=0, keepdims=True)
    sl = jnp.sum(z1 * o, axis=0, keepdims=True)
    la = jnp.sum(jnp.where(mact, o, 0.0), axis=0, keepdims=True)
    lse = jnp.log(se)

    logp_ref[...] = (la - lse).reshape(bb)
    ent_ref[...] = (lse - sl / se).reshape(bb)
    val_ref[...] = o[_E * _A:_E * _A + 1, :].reshape(bb)


def kernel(x, action, W1, b1, W2, b2, Wh, bh, Wc1, bc1, Wc2, bc2, Wc3, bc3):
    B, OBS = x.shape
    H, A, E = _H, _A, _E

    xt = x.T
    act = action.astype(jnp.int32)

    w1t = jnp.concatenate([W1.T, Wc1.T], axis=0)
    b1c = jnp.concatenate([b1, bc1])[:, None]
    z64 = jnp.zeros((H, H), jnp.float32)
    w2t = jnp.block([[W2.T, z64], [z64, Wc2.T]])
    b2c = jnp.concatenate([b2, bc2])[:, None]
    wh_t = jnp.transpose(Wh, (1, 0, 2)).reshape(H, E * A).T
    w3t = jnp.concatenate([
        jnp.concatenate([wh_t, jnp.zeros((E * A, H), jnp.float32)], axis=1),
        jnp.concatenate([jnp.zeros((1, H), jnp.float32), Wc3.T], axis=1),
    ], axis=0)
    b3c = jnp.concatenate([bh.reshape(E * A), bc3])[:, None]

    nb = B // _BB
    const = lambda i: (0, 0)
    out_shape = [jax.ShapeDtypeStruct((B,), jnp.float32)] * 3
    logp, ent, val = pl.pallas_call(
        _fused_body,
        grid=(nb,),
        in_specs=[
            pl.BlockSpec((OBS, _BB), lambda i: (0, i)),
            pl.BlockSpec((_BB,), lambda i: (i,)),
            pl.BlockSpec((2 * H, OBS), const),
            pl.BlockSpec((2 * H, 1), const),
            pl.BlockSpec((2 * H, 2 * H), const),
            pl.BlockSpec((2 * H, 1), const),
            pl.BlockSpec((E * A + 1, 2 * H), const),
            pl.BlockSpec((E * A + 1, 1), const),
        ],
        out_specs=[pl.BlockSpec((_BB,), lambda i: (i,))] * 3,
        out_shape=out_shape,
        compiler_params=pltpu.CompilerParams(
            dimension_semantics=("parallel",),
        ),
        name="agent_fused_t",
    )(xt, act, w1t, b1c, w2t, b2c, w3t, b3c)

    return action, logp, ent, val.reshape(B, 1)


# speedup vs baseline: 8.7327x; 1.0586x over previous
import jax
import jax.numpy as jnp
from jax.experimental import pallas as pl
from jax.experimental.pallas import tpu as pltpu

_H, _A, _E = 64, 5, 3
_BB = 16384


def _fused_body(xt_ref, act_ref, w1_ref, b1_ref, w2_ref, b2_ref, w3_ref,
                b3_ref, logp_ref, ent_ref, val_ref):
    bb = xt_ref.shape[1]
    xt = xt_ref[...]
    h = jnp.tanh(jnp.dot(w1_ref[...], xt,
                         preferred_element_type=jnp.float32) + b1_ref[...])
    g = jnp.tanh(jnp.dot(w2_ref[...], h,
                         preferred_element_type=jnp.float32) + b2_ref[...])
    o = jnp.dot(w3_ref[...], g,
                preferred_element_type=jnp.float32) + b3_ref[...]

    x0, x1, x2 = xt[0:1, :], xt[1:2, :], xt[2:3, :]
    is0 = (x0 >= x1) & (x0 >= x2)
    is1 = jnp.logical_not(is0) & (x1 >= x2)
    ev5 = jnp.where(is0, 0, jnp.where(is1, _A, 2 * _A))

    ki = jax.lax.broadcasted_iota(jnp.int32, (_E * _A + 1, bb), 0)
    msel = (ki >= ev5) & (ki < ev5 + _A)
    mact = ki == ev5 + act_ref[...].reshape(1, bb)

    eo = jnp.exp(o)
    z1 = jnp.where(msel, eo, 0.0)
    se = jnp.sum(z1, axis=0, keepdims=True)
    sl = jnp.sum(z1 * o, axis=0, keepdims=True)
    la = jnp.sum(jnp.where(mact, o, 0.0), axis=0, keepdims=True)
    lse = jnp.log(se)

    logp_ref[...] = (la - lse).reshape(bb)
    ent_ref[...] = (lse - sl / se).reshape(bb)
    val_ref[...] = o[_E * _A:_E * _A + 1, :].reshape(bb)


def kernel(x, action, W1, b1, W2, b2, Wh, bh, Wc1, bc1, Wc2, bc2, Wc3, bc3):
    B, OBS = x.shape
    H, A, E = _H, _A, _E

    xt = x.T
    act = action.astype(jnp.int32)

    w1t = jnp.concatenate([W1.T, Wc1.T], axis=0)
    b1c = jnp.concatenate([b1, bc1])[:, None]
    z64 = jnp.zeros((H, H), jnp.float32)
    w2t = jnp.block([[W2.T, z64], [z64, Wc2.T]])
    b2c = jnp.concatenate([b2, bc2])[:, None]
    wh_t = jnp.transpose(Wh, (1, 0, 2)).reshape(H, E * A).T
    w3t = jnp.concatenate([
        jnp.concatenate([wh_t, jnp.zeros((E * A, H), jnp.float32)], axis=1),
        jnp.concatenate([jnp.zeros((1, H), jnp.float32), Wc3.T], axis=1),
    ], axis=0)
    b3c = jnp.concatenate([bh.reshape(E * A), bc3])[:, None]

    nb = B // _BB
    const = lambda i: (0, 0)
    out_shape = [jax.ShapeDtypeStruct((B,), jnp.float32)] * 3
    logp, ent, val = pl.pallas_call(
        _fused_body,
        grid=(nb,),
        in_specs=[
            pl.BlockSpec((OBS, _BB), lambda i: (0, i)),
            pl.BlockSpec((_BB,), lambda i: (i,)),
            pl.BlockSpec((2 * H, OBS), const),
            pl.BlockSpec((2 * H, 1), const),
            pl.BlockSpec((2 * H, 2 * H), const),
            pl.BlockSpec((2 * H, 1), const),
            pl.BlockSpec((E * A + 1, 2 * H), const),
            pl.BlockSpec((E * A + 1, 1), const),
        ],
        out_specs=[pl.BlockSpec((_BB,), lambda i: (i,))] * 3,
        out_shape=out_shape,
        compiler_params=pltpu.CompilerParams(
            dimension_semantics=("parallel",),
            vmem_limit_bytes=56 * 1024 * 1024,
        ),
        name="agent_fused_t",
    )(xt, act, w1t, b1c, w2t, b2c, w3t, b3c)

    return action, logp, ent, val.reshape(B, 1)


# speedup vs baseline: 9.3738x; 1.0734x over previous
import jax
import jax.numpy as jnp
from jax.experimental import pallas as pl
from jax.experimental.pallas import tpu as pltpu

_H, _A, _E = 64, 5, 3
_BB = 16384
_NROW = _E * _A + 1


def _fused_body(xt_ref, act_ref, wb_ref, act_out_ref, logp_ref, ent_ref,
                val_ref):
    bb = xt_ref.shape[1]
    w1 = wb_ref[:, 0:194]
    w2 = wb_ref[:, 256:384]
    w3 = wb_ref[0:_NROW, 384:512]
    b1 = wb_ref[:, 512:513]
    b2 = wb_ref[:, 513:514]
    b3 = wb_ref[0:_NROW, 514:515]

    xt = xt_ref[...]
    h = jnp.tanh(jnp.dot(w1, xt, preferred_element_type=jnp.float32) + b1)
    g = jnp.tanh(jnp.dot(w2, h, preferred_element_type=jnp.float32) + b2)
    o = jnp.dot(w3, g, preferred_element_type=jnp.float32) + b3

    x0, x1, x2 = xt[0:1, :], xt[1:2, :], xt[2:3, :]
    is0 = (x0 >= x1) & (x0 >= x2)
    is1 = jnp.logical_not(is0) & (x1 >= x2)
    ev5 = jnp.where(is0, 0, jnp.where(is1, _A, 2 * _A))

    act = act_ref[...]
    ki = jax.lax.broadcasted_iota(jnp.int32, (_NROW, bb), 0)
    msel = (ki >= ev5) & (ki < ev5 + _A)
    mact = ki == ev5 + act.reshape(1, bb)

    eo = jnp.exp(o)
    z1 = jnp.where(msel, eo, 0.0)
    se = jnp.sum(z1, axis=0, keepdims=True)
    sl = jnp.sum(z1 * o, axis=0, keepdims=True)
    la = jnp.sum(jnp.where(mact, o, 0.0), axis=0, keepdims=True)
    lse = jnp.log(se)

    act_out_ref[...] = act
    logp_ref[...] = (la - lse).reshape(bb)
    ent_ref[...] = (lse - sl / se).reshape(bb)
    val_ref[...] = o[_E * _A:_E * _A + 1, :].reshape(bb)


def kernel(x, action, W1, b1, W2, b2, Wh, bh, Wc1, bc1, Wc2, bc2, Wc3, bc3):
    B, OBS = x.shape
    H, A, E = _H, _A, _E

    xt = x.T
    act = action.astype(jnp.int32)

    z64 = jnp.zeros((H, H), jnp.float32)
    w1t = jnp.concatenate([W1.T, Wc1.T], axis=0)
    w1p = jnp.pad(w1t, ((0, 0), (0, 256 - OBS)))
    w2t = jnp.block([[W2.T, z64], [z64, Wc2.T]])
    wh_t = jnp.transpose(Wh, (1, 0, 2)).reshape(H, E * A).T
    w3t = jnp.concatenate([
        jnp.concatenate([wh_t, jnp.zeros((E * A, H), jnp.float32)], axis=1),
        jnp.concatenate([jnp.zeros((1, H), jnp.float32), Wc3.T], axis=1),
    ], axis=0)
    w3p = jnp.pad(w3t, ((0, 128 - _NROW), (0, 0)))
    b1c = jnp.concatenate([b1, bc1])[:, None]
    b2c = jnp.concatenate([b2, bc2])[:, None]
    b3c = jnp.pad(jnp.concatenate([bh.reshape(E * A), bc3]),
                  (0, 128 - _NROW))[:, None]
    blob = jnp.concatenate([w1p, w2t, w3p, b1c, b2c, b3c], axis=1)

    nb = B // _BB
    out_shape = [
        jax.ShapeDtypeStruct((B,), jnp.int32),
        jax.ShapeDtypeStruct((B,), jnp.float32),
        jax.ShapeDtypeStruct((B,), jnp.float32),
        jax.ShapeDtypeStruct((B,), jnp.float32),
    ]
    act_out, logp, ent, val = pl.pallas_call(
        _fused_body,
        grid=(nb,),
        in_specs=[
            pl.BlockSpec((OBS, _BB), lambda i: (0, i)),
            pl.BlockSpec((_BB,), lambda i: (i,)),
            pl.BlockSpec((2 * H, 515), lambda i: (0, 0)),
        ],
        out_specs=[pl.BlockSpec((_BB,), lambda i: (i,))] * 4,
        out_shape=out_shape,
        compiler_params=pltpu.CompilerParams(
            dimension_semantics=("parallel",),
            vmem_limit_bytes=56 * 1024 * 1024,
        ),
        name="agent_fused_t",
    )(xt, act, blob)

    return act_out, logp, ent, val.reshape(B, 1)


# speedup vs baseline: 11.0115x; 1.1747x over previous
import jax
import jax.numpy as jnp
from jax.experimental import pallas as pl
from jax.experimental.pallas import tpu as pltpu

_H, _A, _E = 64, 5, 3
_BB = 16384
_NROW = _E * _A + 1


def _fused_body(xt_ref, act_ref, w1t_ref, b1_ref, w2_ref, b2_ref, whp_ref,
                bh_ref, wc1t_ref, bc1_ref, wc2_ref, bc2_ref, wc3t_ref,
                bc3_ref, act_out_ref, logp_ref, ent_ref, val_ref, ws):
    bb = xt_ref.shape[1]

    @pl.when(pl.program_id(0) == 0)
    def _pack():
        z = jnp.zeros((_H, _H), jnp.float32)
        ws[0:_H, 0:194] = w1t_ref[...]
        ws[_H:2 * _H, 0:194] = wc1t_ref[...]
        ws[0:_H, 256:320] = w2_ref[...].T
        ws[0:_H, 320:384] = z
        ws[_H:2 * _H, 256:320] = z
        ws[_H:2 * _H, 320:384] = wc2_ref[...].T
        whp = whp_ref[...]
        for e in range(_E):
            ws[_A * e:_A * e + _A, 384:448] = whp[:, e, :]
            ws[_A * e:_A * e + _A, 768:769] = bh_ref[e:e + 1, :].T
        ws[0:_E * _A, 448:512] = jnp.zeros((_E * _A, _H), jnp.float32)
        ws[_E * _A:_NROW, 384:448] = jnp.zeros((1, _H), jnp.float32)
        ws[_E * _A:_NROW, 448:512] = wc3t_ref[...]
        ws[0:_H, 512:513] = b1_ref[...].T
        ws[_H:2 * _H, 512:513] = bc1_ref[...].T
        ws[0:_H, 640:641] = b2_ref[...].T
        ws[_H:2 * _H, 640:641] = bc2_ref[...].T
        ws[_E * _A:_NROW, 768:769] = bc3_ref[...]

    xt = xt_ref[...]
    h = jnp.tanh(jnp.dot(ws[:, 0:194], xt,
                         preferred_element_type=jnp.float32) + ws[:, 512:513])
    g = jnp.tanh(jnp.dot(ws[:, 256:384], h,
                         preferred_element_type=jnp.float32) + ws[:, 640:641])
    o = (jnp.dot(ws[0:_NROW, 384:512], g, preferred_element_type=jnp.float32)
         + ws[0:_NROW, 768:769])

    x0, x1, x2 = xt[0:1, :], xt[1:2, :], xt[2:3, :]
    is0 = (x0 >= x1) & (x0 >= x2)
    is1 = jnp.logical_not(is0) & (x1 >= x2)
    ev5 = jnp.where(is0, 0, jnp.where(is1, _A, 2 * _A))

    act = act_ref[...]
    ki = jax.lax.broadcasted_iota(jnp.int32, (_NROW, bb), 0)
    msel = (ki >= ev5) & (ki < ev5 + _A)
    mact = ki == ev5 + act.reshape(1, bb)

    eo = jnp.exp(o)
    z1 = jnp.where(msel, eo, 0.0)
    se = jnp.sum(z1, axis=0, keepdims=True)
    sl = jnp.sum(z1 * o, axis=0, keepdims=True)
    la = jnp.sum(jnp.where(mact, o, 0.0), axis=0, keepdims=True)
    lse = jnp.log(se)

    act_out_ref[...] = act
    logp_ref[...] = (la - lse).reshape(bb)
    ent_ref[...] = (lse - sl / se).reshape(bb)
    val_ref[...] = o[_E * _A:_E * _A + 1, :].reshape(bb)


def kernel(x, action, W1, b1, W2, b2, Wh, bh, Wc1, bc1, Wc2, bc2, Wc3, bc3):
    B, OBS = x.shape
    H, A, E = _H, _A, _E

    xt = x.T
    act = action.astype(jnp.int32)

    nb = B // _BB
    full = lambda *dims: (lambda i: tuple(0 for _ in dims))
    out_shape = [
        jax.ShapeDtypeStruct((B,), jnp.int32),
        jax.ShapeDtypeStruct((B,), jnp.float32),
        jax.ShapeDtypeStruct((B,), jnp.float32),
        jax.ShapeDtypeStruct((B,), jnp.float32),
    ]
    act_out, logp, ent, val = pl.pallas_call(
        _fused_body,
        grid=(nb,),
        in_specs=[
            pl.BlockSpec((OBS, _BB), lambda i: (0, i)),
            pl.BlockSpec((_BB,), lambda i: (i,)),
            pl.BlockSpec((H, OBS), full(0, 0)),
            pl.BlockSpec((1, H), full(0, 0)),
            pl.BlockSpec((H, H), full(0, 0)),
            pl.BlockSpec((1, H), full(0, 0)),
            pl.BlockSpec((A, E, H), full(0, 0, 0)),
            pl.BlockSpec((E, A), full(0, 0)),
            pl.BlockSpec((H, OBS), full(0, 0)),
            pl.BlockSpec((1, H), full(0, 0)),
            pl.BlockSpec((H, H), full(0, 0)),
            pl.BlockSpec((1, H), full(0, 0)),
            pl.BlockSpec((1, H), full(0, 0)),
            pl.BlockSpec((1, 1), full(0, 0)),
        ],
        out_specs=[pl.BlockSpec((_BB,), lambda i: (i,))] * 4,
        out_shape=out_shape,
        scratch_shapes=[pltpu.VMEM((2 * H, 1024), jnp.float32)],
        compiler_params=pltpu.CompilerParams(
            dimension_semantics=("arbitrary",),
            vmem_limit_bytes=56 * 1024 * 1024,
        ),
        name="agent_fused_t",
    )(xt, act, W1.T, b1.reshape(1, H), W2, b2.reshape(1, H),
      Wh.transpose(2, 0, 1), bh, Wc1.T, bc1.reshape(1, H), Wc2,
      bc2.reshape(1, H), Wc3.T, bc3.reshape(1, 1))

    return act_out, logp, ent, val.reshape(B, 1)
